# baseline (speedup 1.0000x reference)
import jax
import jax.numpy as jnp
from jax import lax
from jax.experimental import pallas as pl
from jax.experimental.pallas import tpu as pltpu
from jax.experimental.pallas import tpu_sc as plsc

VOCAB = 1000
EMB = 32
BATCH = 16384
SEQ = 200

B = BATCH * SEQ
NC, NS = 2, 16
NW = NC * NS
IDX_MINOR = 128
CHUNK_DMAS = 8
CHUNK = IDX_MINOR * CHUNK_DMAS
ROWS_PER_W = B // NW
ITERS = ROWS_PER_W // CHUNK
X2D_ROWS_PER_W = ROWS_PER_W // IDX_MINOR
NBUF = 3

assert (ITERS - 4) % NBUF == 0


def _emb_kernel(x2d_hbm, table_hbm, out_hbm,
                idx0, idx1, idx2, rows0, rows1, rows2,
                si0, si1, si2, sg0, sg1, sg2, so0, so1, so2):
    idx = [idx0, idx1, idx2]
    rows = [rows0, rows1, rows2]
    si = [si0, si1, si2]
    sg = [sg0, sg1, sg2]
    so = [so0, so1, so2]

    wid = lax.axis_index("s") * NC + lax.axis_index("c")
    x2d_base = wid * X2D_ROWS_PER_W
    out_base = wid * ROWS_PER_W

    def fire_idx(g, b):
        pltpu.async_copy(
            x2d_hbm.at[pl.ds(x2d_base + g * CHUNK_DMAS, CHUNK_DMAS)],
            idx[b], si[b])

    def wait_idx(b):
        pltpu.make_async_copy(
            x2d_hbm.at[pl.ds(x2d_base, CHUNK_DMAS)], idx[b], si[b]).wait()

    def fire_gath(b):
        for j in range(CHUNK_DMAS):
            pltpu.async_copy(table_hbm.at[idx[b].at[j]],
                             rows[b].at[pl.ds(j * IDX_MINOR, IDX_MINOR)],
                             sg[b])

    def wait_gath(b):
        pltpu.make_async_copy(
            out_hbm.at[pl.ds(out_base, CHUNK)], rows[b], sg[b]).wait()

    def fire_out(g, b):
        pltpu.async_copy(rows[b],
                         out_hbm.at[pl.ds(out_base + g * CHUNK, CHUNK)],
                         so[b])

    def wait_out(b):
        pltpu.make_async_copy(
            rows[b], out_hbm.at[pl.ds(out_base, CHUNK)], so[b]).wait()

    fire_idx(0, 0)
    fire_idx(1, 1)
    fire_idx(2, 2)
    wait_idx(0)
    fire_gath(0)
    wait_idx(1)
    fire_gath(1)
    wait_gath(0)
    fire_out(0, 0)
    fire_idx(3, 0)
    wait_idx(2)
    fire_gath(2)

    def body(t, carry):
        for b_off in range(NBUF):
            g = NBUF * t + 1 + b_off
            b = (1 + b_off) % NBUF
            b2 = b_off % NBUF
            wait_gath(b)
            fire_out(g, b)
            fire_idx(g + NBUF, b)
            wait_idx(b2)
            wait_out(b2)
            fire_gath(b2)
        return carry

    lax.fori_loop(0, (ITERS - 4) // NBUF, body, 0)

    wait_gath(1)
    fire_out(ITERS - 3, 1)
    wait_idx(0)
    wait_out(0)
    fire_gath(0)
    wait_gath(2)
    fire_out(ITERS - 2, 2)
    wait_gath(0)
    fire_out(ITERS - 1, 0)
    wait_out(1)
    wait_out(2)
    wait_out(0)


@jax.jit
def _run(x2d, table):
    mesh = plsc.VectorSubcoreMesh(core_axis_name="c", subcore_axis_name="s")
    return pl.kernel(
        _emb_kernel,
        mesh=mesh,
        out_type=jax.ShapeDtypeStruct((B, EMB), jnp.float32),
        scratch_types=(
            [pltpu.VMEM((CHUNK_DMAS, IDX_MINOR), jnp.int32)] * NBUF
            + [pltpu.VMEM((CHUNK, EMB), jnp.float32)] * NBUF
            + [pltpu.SemaphoreType.DMA] * (3 * NBUF)
        ),
        compiler_params=pltpu.CompilerParams(use_tc_tiling_on_sc=False),
    )(x2d, table)


def kernel(x, table):
    x2d = x.reshape(B // IDX_MINOR, IDX_MINOR).astype(jnp.int32)
    out = _run(x2d, table)
    return out.reshape(BATCH, SEQ, EMB)


# speedup vs baseline: 1.0086x; 1.0086x over previous
import jax
import jax.numpy as jnp
from jax import lax
from jax.experimental import pallas as pl
from jax.experimental.pallas import tpu as pltpu
from jax.experimental.pallas import tpu_sc as plsc

VOCAB = 1000
EMB = 32
BATCH = 16384
SEQ = 200

B = BATCH * SEQ
NC, NS, L = 2, 16, 16
NW = NC * NS
CHUNK = 1024
ROWS_PER_W = B // NW
ITERS = ROWS_PER_W // CHUNK
GROUPS = CHUNK // L

assert ITERS % 2 == 0 and ITERS >= 4


def _bcast_lane(v, i):
    return lax.gather(
        v,
        jnp.full((L, 1), i, jnp.int32),
        lax.GatherDimensionNumbers(
            offset_dims=(), collapsed_slice_dims=(0,), start_index_map=(0,)),
        (1,),
        mode=lax.GatherScatterMode.PROMISE_IN_BOUNDS)


def _emb_kernel(x_hbm, table_hbm, out_hbm,
                table_v, idx0, idx1, rows0, rows1, si0, si1, so0, so1):
    idx = [idx0, idx1]
    rows = [rows0, rows1]
    si = [si0, si1]
    so = [so0, so1]

    wid = lax.axis_index("s") * NC + lax.axis_index("c")
    base = wid * ROWS_PER_W

    pltpu.sync_copy(table_hbm, table_v)

    col0 = lax.iota(jnp.int32, L)
    col1 = col0 + L

    def fire_idx(c, b):
        pltpu.async_copy(x_hbm.at[pl.ds(base + c * CHUNK, CHUNK)], idx[b],
                         si[b])

    def wait_idx(b):
        pltpu.make_async_copy(x_hbm.at[pl.ds(base, CHUNK)], idx[b],
                              si[b]).wait()

    def fire_out(c, b):
        pltpu.async_copy(rows[b],
                         out_hbm.at[pl.ds(base + c * CHUNK, CHUNK)], so[b])

    def wait_out(b):
        pltpu.make_async_copy(rows[b], out_hbm.at[pl.ds(base, CHUNK)],
                              so[b]).wait()

    def compute_chunk(b):
        def grp(k, carry):
            idxv = idx[b][pl.ds(k * L, L)] * EMB
            for i in range(L):
                a0 = _bcast_lane(idxv, i) + col0
                row = k * L + i
                rows[b][row, pl.ds(0, L)] = plsc.load_gather(table_v, [a0])
                rows[b][row, pl.ds(L, L)] = plsc.load_gather(table_v,
                                                             [a0 + L])
            return carry
        lax.fori_loop(0, GROUPS, grp, 0)

    fire_idx(0, 0)
    fire_idx(1, 1)
    wait_idx(0)
    compute_chunk(0)
    fire_out(0, 0)
    fire_idx(2, 0)
    wait_idx(1)
    compute_chunk(1)
    fire_out(1, 1)
    fire_idx(3, 1)

    def body(t, carry):
        for b in range(2):
            c = 2 * t + b
            wait_idx(b)
            wait_out(b)
            compute_chunk(b)
            fire_out(c, b)
            fire_idx(c + 2, b)
        return carry

    lax.fori_loop(1, ITERS // 2 - 1, body, 0)

    for b in range(2):
        c = ITERS - 2 + b
        wait_idx(b)
        wait_out(b)
        compute_chunk(b)
        fire_out(c, b)
    wait_out(0)
    wait_out(1)


@jax.jit
def _run(x_flat, table):
    mesh = plsc.VectorSubcoreMesh(core_axis_name="c", subcore_axis_name="s")
    return pl.kernel(
        _emb_kernel,
        mesh=mesh,
        out_type=jax.ShapeDtypeStruct((B, EMB), jnp.float32),
        scratch_types=(
            [pltpu.VMEM((VOCAB * EMB,), jnp.float32)]
            + [pltpu.VMEM((CHUNK,), jnp.int32)] * 2
            + [pltpu.VMEM((CHUNK, EMB), jnp.float32)] * 2
            + [pltpu.SemaphoreType.DMA] * 4
        ),
        compiler_params=pltpu.CompilerParams(use_tc_tiling_on_sc=False,
                                             needs_layout_passes=False),
    )(x_flat, table)


def kernel(x, table):
    x_flat = x.reshape(B).astype(jnp.int32)
    out = _run(x_flat, table.reshape(VOCAB * EMB))
    return out.reshape(BATCH, SEQ, EMB)
